# baseline (speedup 1.0000x reference)
import functools

import jax
import jax.numpy as jnp
from jax import lax
from jax.experimental import pallas as pl
from jax.experimental.pallas import tpu as pltpu
from jax.experimental.pallas import tpu_sc as plsc

_D = 64
_NC, _NS = 2, 16
_NW = _NC * _NS
_GB = 8
_NB = 4
_HP = 56
_DP = 128


def _embed_body(idx_hbm, table_hbm, out_hbm, idx_v, *rest):
    bufs = rest[:_NB]
    gsems = rest[_NB:2 * _NB]
    wsems = rest[2 * _NB:3 * _NB]
    wid = lax.axis_index("s") * _NC + lax.axis_index("c")
    bpw = idx_v.shape[0]
    hist = idx_v.shape[1]
    ngrp = bpw // _GB
    base_b = wid * bpw
    pltpu.sync_copy(idx_hbm.at[pl.ds(base_b, bpw)], idx_v)

    def fire(g, b):
        for k in range(_GB):
            pltpu.async_copy(table_hbm.at[idx_v.at[g * _GB + k]],
                             bufs[b].at[k], gsems[b])

    def drain_gathers(b):
        pltpu.make_async_copy(out_hbm.at[pl.ds(0, _GB),
                                         pl.ds(0, hist), pl.ds(0, _D)],
                              bufs[b], gsems[b]).wait()

    def fire_write(g, b):
        pltpu.async_copy(bufs[b],
                         out_hbm.at[pl.ds(base_b + g * _GB, _GB),
                                    pl.ds(0, hist), pl.ds(0, _D)], wsems[b])

    def wait_write(b):
        pltpu.make_async_copy(out_hbm.at[pl.ds(0, _GB),
                                         pl.ds(0, hist), pl.ds(0, _D)],
                              bufs[b], wsems[b]).wait()

    for g in range(_NB - 1):
        fire(g, g)

    def step(i, carry):
        for b in range(_NB):
            g = _NB * i + b
            drain_gathers(b)
            fire_write(g, b)
            j = g + _NB - 1
            jb = (_NB - 1 + b) % _NB
            @pl.when(j < ngrp)
            def _():
                @pl.when(j >= _NB)
                def _():
                    wait_write(jb)
                fire(j, jb)
        return carry

    lax.fori_loop(0, ngrp // _NB, step, 0)
    for b in range(_NB):
        wait_write(b)


def kernel(x, embed_mat):
    b, h = x.shape
    n, d = embed_mat.shape
    bpw = b // _NW
    t_lin = jax.lax.optimization_barrier(embed_mat.reshape(n // 2, 2 * d))
    table = t_lin.reshape(n, d)
    mesh = plsc.VectorSubcoreMesh(core_axis_name="c", subcore_axis_name="s",
                                  num_cores=_NC, num_subcores=_NS)
    y3 = pl.kernel(
        _embed_body,
        out_type=jax.ShapeDtypeStruct((b, _HP, _DP), jnp.float32),
        mesh=mesh,
        scratch_types=[
            pltpu.VMEM((bpw, h), jnp.int32),
            *[pltpu.VMEM((_GB, h, _D), jnp.float32) for _ in range(_NB)],
            *[pltpu.SemaphoreType.DMA for _ in range(2 * _NB)],
        ],
        compiler_params=pltpu.CompilerParams(use_tc_tiling_on_sc=False),
    )(x.astype(jnp.int32), table)
    return y3[:, :h, :_D]


# speedup vs baseline: 1.0007x; 1.0007x over previous
import functools

import jax
import jax.numpy as jnp
from jax import lax
from jax.experimental import pallas as pl
from jax.experimental.pallas import tpu as pltpu
from jax.experimental.pallas import tpu_sc as plsc

_D = 64
_NC, _NS = 2, 16
_NW = _NC * _NS
_GB = 8
_NB = 4
_HP = 56
_DP = 128


def _embed_body(idx_hbm, table_hbm, out_hbm, idx_v, *rest):
    bufs = rest[:_NB]
    gsems = rest[_NB:2 * _NB]
    wsems = rest[2 * _NB:3 * _NB]
    wid = lax.axis_index("s") * _NC + lax.axis_index("c")
    bpw = idx_v.shape[0]
    hist = idx_v.shape[1]
    ngrp = bpw // _GB
    base_b = wid * bpw
    pltpu.sync_copy(idx_hbm.at[pl.ds(base_b, bpw)], idx_v)

    def fire(g, b):
        for k in range(_GB):
            pltpu.async_copy(table_hbm.at[idx_v.at[g * _GB + k]],
                             bufs[b].at[k], gsems[b])

    def drain_gathers(b):
        pltpu.make_async_copy(out_hbm.at[pl.ds(0, _GB),
                                         pl.ds(0, hist), pl.ds(0, _D)],
                              bufs[b], gsems[b]).wait()

    def fire_write(g, b):
        pltpu.async_copy(bufs[b],
                         out_hbm.at[pl.ds(base_b + g * _GB, _GB),
                                    pl.ds(0, hist), pl.ds(0, _D)], wsems[b])

    def wait_write(b):
        pltpu.make_async_copy(out_hbm.at[pl.ds(0, _GB),
                                         pl.ds(0, hist), pl.ds(0, _D)],
                              bufs[b], wsems[b]).wait()

    for g in range(_NB - 1):
        fire(g, g)

    def step(i, carry):
        for b in range(_NB):
            g = _NB * i + b
            drain_gathers(b)
            fire_write(g, b)
            j = g + _NB - 1
            jb = (_NB - 1 + b) % _NB
            @pl.when(j < ngrp)
            def _():
                @pl.when(j >= _NB)
                def _():
                    wait_write(jb)
                fire(j, jb)
        return carry

    lax.fori_loop(0, ngrp // _NB, step, 0)
    for b in range(_NB):
        wait_write(b)


def kernel(x, embed_mat):
    b, h = x.shape
    bpw = b // _NW
    mesh = plsc.VectorSubcoreMesh(core_axis_name="c", subcore_axis_name="s",
                                  num_cores=_NC, num_subcores=_NS)
    y3 = pl.kernel(
        _embed_body,
        out_type=jax.ShapeDtypeStruct((b, _HP, _DP), jnp.float32),
        mesh=mesh,
        scratch_types=[
            pltpu.VMEM((bpw, h), jnp.int32),
            *[pltpu.VMEM((_GB, h, _D), jnp.float32) for _ in range(_NB)],
            *[pltpu.SemaphoreType.DMA for _ in range(2 * _NB)],
        ],
        compiler_params=pltpu.CompilerParams(use_tc_tiling_on_sc=False),
    )(x.astype(jnp.int32), embed_mat)
    return y3[:, :h, :_D]


# speedup vs baseline: 1.0021x; 1.0014x over previous
import functools

import jax
import jax.numpy as jnp
from jax import lax
from jax.experimental import pallas as pl
from jax.experimental.pallas import tpu as pltpu
from jax.experimental.pallas import tpu_sc as plsc

_D = 64
_NC, _NS = 2, 16
_NW = _NC * _NS
_GB = 4
_NB = 8
_HP = 56
_DP = 128


def _embed_body(idx_hbm, table_hbm, out_hbm, idx_v, *rest):
    bufs = rest[:_NB]
    gsems = rest[_NB:2 * _NB]
    wsems = rest[2 * _NB:3 * _NB]
    wid = lax.axis_index("s") * _NC + lax.axis_index("c")
    bpw = idx_v.shape[0]
    hist = idx_v.shape[1]
    ngrp = bpw // _GB
    base_b = wid * bpw
    pltpu.sync_copy(idx_hbm.at[pl.ds(base_b, bpw)], idx_v)

    def fire(g, b):
        for k in range(_GB):
            pltpu.async_copy(table_hbm.at[idx_v.at[g * _GB + k]],
                             bufs[b].at[k], gsems[b])

    def drain_gathers(b):
        pltpu.make_async_copy(out_hbm.at[pl.ds(0, _GB),
                                         pl.ds(0, hist), pl.ds(0, _D)],
                              bufs[b], gsems[b]).wait()

    def fire_write(g, b):
        pltpu.async_copy(bufs[b],
                         out_hbm.at[pl.ds(base_b + g * _GB, _GB),
                                    pl.ds(0, hist), pl.ds(0, _D)], wsems[b])

    def wait_write(b):
        pltpu.make_async_copy(out_hbm.at[pl.ds(0, _GB),
                                         pl.ds(0, hist), pl.ds(0, _D)],
                              bufs[b], wsems[b]).wait()

    for g in range(_NB - 1):
        fire(g, g)

    def step(i, carry):
        for b in range(_NB):
            g = _NB * i + b
            drain_gathers(b)
            fire_write(g, b)
            j = g + _NB - 1
            jb = (_NB - 1 + b) % _NB
            @pl.when(j < ngrp)
            def _():
                @pl.when(j >= _NB)
                def _():
                    wait_write(jb)
                fire(j, jb)
        return carry

    lax.fori_loop(0, ngrp // _NB, step, 0)
    for b in range(_NB):
        wait_write(b)


def kernel(x, embed_mat):
    b, h = x.shape
    bpw = b // _NW
    mesh = plsc.VectorSubcoreMesh(core_axis_name="c", subcore_axis_name="s",
                                  num_cores=_NC, num_subcores=_NS)
    y3 = pl.kernel(
        _embed_body,
        out_type=jax.ShapeDtypeStruct((b, _HP, _DP), jnp.float32),
        mesh=mesh,
        scratch_types=[
            pltpu.VMEM((bpw, h), jnp.int32),
            *[pltpu.VMEM((_GB, h, _D), jnp.float32) for _ in range(_NB)],
            *[pltpu.SemaphoreType.DMA for _ in range(2 * _NB)],
        ],
        compiler_params=pltpu.CompilerParams(use_tc_tiling_on_sc=False),
    )(x.astype(jnp.int32), embed_mat)
    return y3[:, :h, :_D]
